# baseline (speedup 1.0000x reference)
import functools

import jax
import jax.numpy as jnp
from jax import lax
from jax.experimental import pallas as pl
from jax.experimental.pallas import tpu as pltpu
from jax.experimental.pallas import tpu_sc as plsc

_BATCH = 16384
_FEAT = 64
_LANES = 16

_NC = 2
_NS = 16
_NW = _NC * _NS
_ROWS_W = _BATCH // _NW
_IDX_CHUNK = 128
_N_CHUNKS = _ROWS_W // _IDX_CHUNK
_LBL_ROWS_W = _ROWS_W // _IDX_CHUNK
_GROUPS_Q = _IDX_CHUNK // _LANES

_PACK_BN = 8192
_QTR = _PACK_BN // 4
_BN_LOG = 13
_QTR_LOG = 11


def _bf16_bits(u):
    lsb = jnp.bitwise_and(lax.shift_right_logical(u, jnp.uint32(16)),
                          jnp.uint32(1))
    return lax.shift_right_logical(u + jnp.uint32(0x7FFF) + lsb,
                                   jnp.uint32(16))


def _pack_body(in_ref, out_ref):
    x = in_ref[...]
    parts = []
    for q in range(4):
        t = jnp.transpose(x[:, q * _QTR:(q + 1) * _QTR])
        ua = lax.bitcast_convert_type(t[:, 0:32], jnp.uint32)
        ub = lax.bitcast_convert_type(t[:, 32:64], jnp.uint32)
        packed = jnp.bitwise_or(
            _bf16_bits(ua),
            lax.shift_left(_bf16_bits(ub), jnp.uint32(16)))
        parts.append(lax.bitcast_convert_type(packed, jnp.float32))
    out_ref[...] = jnp.concatenate(parts, axis=1)


def _pack(xt):
    n = xt.shape[1]
    grid = (n + _PACK_BN - 1) // _PACK_BN
    return pl.pallas_call(
        _pack_body,
        grid=(grid,),
        in_specs=[pl.BlockSpec((_FEAT, _PACK_BN), lambda i: (0, i))],
        out_specs=pl.BlockSpec((_QTR, 128), lambda i: (i, 0)),
        out_shape=jax.ShapeDtypeStruct((grid * _QTR, 128), jnp.float32),
    )(xt)


@functools.partial(
    pl.kernel,
    mesh=plsc.VectorSubcoreMesh(core_axis_name="c", subcore_axis_name="s"),
    compiler_params=pltpu.CompilerParams(use_tc_tiling_on_sc=True,
                                         needs_layout_passes=False),
    out_type=jax.ShapeDtypeStruct((_NW * _LANES,), jnp.float32),
    scratch_types=[
        pltpu.VMEM((_ROWS_W,), jnp.int32),
        pltpu.VMEM((_N_CHUNKS, _IDX_CHUNK), jnp.int32),
        pltpu.VMEM((_ROWS_W // _LANES, _LANES), jnp.int32),
        pltpu.VMEM((2, _IDX_CHUNK, 128), jnp.float32),
        pltpu.VMEM((2, _IDX_CHUNK, 128), jnp.float32),
        pltpu.VMEM((_LANES,), jnp.float32),
        pltpu.SemaphoreType.DMA,
        pltpu.SemaphoreType.DMA,
    ],
)
def _center_loss_partials(x_hbm, labels_hbm, centers_hbm, out_hbm,
                          idx_v, row_v, off_v, rows_v, x_v, acc_v,
                          sem_x, sem_g):
    wid = lax.axis_index("s") * _NC + lax.axis_index("c")

    pltpu.sync_copy(
        labels_hbm.at[pl.ds(pl.multiple_of(wid * _ROWS_W, 8), _ROWS_W)],
        idx_v)
    for j in range(_N_CHUNKS):
        for k in range(_IDX_CHUNK // _LANES):
            v = idx_v[pl.ds(j * _IDX_CHUNK + k * _LANES, _LANES)]
            blk = lax.shift_right_logical(v, jnp.int32(_BN_LOG))
            row_v[j, pl.ds(k * _LANES, _LANES)] = (
                lax.shift_left(blk, jnp.int32(_QTR_LOG))
                + jnp.bitwise_and(v, jnp.int32(_QTR - 1)))
            off_v[j * (_IDX_CHUNK // _LANES) + k, pl.ds(0, _LANES)] = (
                lax.shift_left(
                    jnp.bitwise_and(
                        lax.shift_right_logical(v, jnp.int32(_QTR_LOG)),
                        jnp.int32(3)),
                    jnp.int32(5)))

    w15 = jnp.bitwise_and(wid, jnp.int32(15))
    xbase = (lax.shift_left(lax.shift_right_logical(wid, jnp.int32(4)),
                            jnp.int32(_QTR_LOG))
             + jnp.bitwise_and(w15, jnp.int32(3)) * jnp.int32(_ROWS_W))
    xoff = pl.multiple_of(
        lax.shift_left(lax.shift_right_logical(w15, jnp.int32(2)),
                       jnp.int32(5)), 32)

    def start_quarter(q):
        slot = q % 2
        xc = pltpu.async_copy(
            x_hbm.at[pl.ds(pl.multiple_of(xbase + q * _IDX_CHUNK, 8),
                           _IDX_CHUNK)],
            x_v.at[slot], sem_x)
        gc = pltpu.async_copy(centers_hbm.at[row_v.at[q]], rows_v.at[slot],
                              sem_g)
        return (xc, gc)

    def compute_quarter(q, acc):
        slot = q % 2

        def group_body(g, acc):
            off16 = off_v[q * _GROUPS_Q + g, pl.ds(0, _LANES)]
            for i in range(_LANES):
                r = g * _LANES + i
                off = pl.multiple_of(off16[i], 32)
                for half in range(2):
                    xv = x_v[slot, r, pl.ds(xoff + half * _LANES, _LANES)]
                    cv = rows_v[slot, r, pl.ds(off + half * _LANES, _LANES)]
                    xb = plsc.bitcast(xv, jnp.bfloat16)
                    cb = plsc.bitcast(cv, jnp.bfloat16)
                    d = xb - cb
                    da, db = plsc.unpack(
                        d, format=plsc.PackFormat.INTERLEAVED)
                    acc = acc + da * da
                    acc = acc + db * db
            return acc
        return lax.fori_loop(0, _GROUPS_Q, group_body, acc)

    acc = jnp.zeros((_LANES,), jnp.float32)
    pend = start_quarter(0)
    for q in range(_N_CHUNKS):
        nxt = start_quarter(q + 1) if q + 1 < _N_CHUNKS else None
        for c in pend:
            c.wait()
        acc = compute_quarter(q, acc)
        pend = nxt

    acc_v[...] = acc
    pltpu.sync_copy(
        acc_v,
        out_hbm.at[pl.ds(pl.multiple_of(wid * _LANES, 8), _LANES)])


def kernel(x, labels, centers):
    xp = _pack(x.T)
    cp = _pack(centers.T)
    labels_r = labels.astype(jnp.int32)
    partials = _center_loss_partials(xp, labels_r, cp)
    return jnp.sum(partials) * (1.0 / _BATCH)


# speedup vs baseline: 1.1226x; 1.1226x over previous
import functools

import jax
import jax.numpy as jnp
from jax import lax
from jax.experimental import pallas as pl
from jax.experimental.pallas import tpu as pltpu
from jax.experimental.pallas import tpu_sc as plsc

_BATCH = 16384
_FEAT = 64
_LANES = 16

_NC = 2
_NS = 16
_NW = _NC * _NS
_ROWS_W = _BATCH // _NW
_IDX_CHUNK = 128
_N_CHUNKS = _ROWS_W // _IDX_CHUNK
_LBL_ROWS_W = _ROWS_W // _IDX_CHUNK
_GROUPS_Q = _IDX_CHUNK // _LANES

_PACK_BN = 8192
_QTR = _PACK_BN // 4
_BN_LOG = 13
_QTR_LOG = 11


def _pack_body(in_ref, out_ref):
    x = in_ref[...]
    parts = []
    for q in range(4):
        t = jnp.transpose(x[:, q * _QTR:(q + 1) * _QTR])
        t16 = t.astype(jnp.bfloat16)
        au = lax.bitcast_convert_type(t16[:, 0:32],
                                      jnp.uint16).astype(jnp.uint32)
        bu = lax.bitcast_convert_type(t16[:, 32:64],
                                      jnp.uint16).astype(jnp.uint32)
        packed = jnp.bitwise_or(au, lax.shift_left(bu, jnp.uint32(16)))
        parts.append(lax.bitcast_convert_type(packed, jnp.float32))
    out_ref[...] = jnp.concatenate(parts, axis=1)


def _pack(xt):
    n = xt.shape[1]
    grid = (n + _PACK_BN - 1) // _PACK_BN
    return pl.pallas_call(
        _pack_body,
        grid=(grid,),
        in_specs=[pl.BlockSpec((_FEAT, _PACK_BN), lambda i: (0, i))],
        out_specs=pl.BlockSpec((_QTR, 128), lambda i: (i, 0)),
        out_shape=jax.ShapeDtypeStruct((grid * _QTR, 128), jnp.float32),
    )(xt)


@functools.partial(
    pl.kernel,
    mesh=plsc.VectorSubcoreMesh(core_axis_name="c", subcore_axis_name="s"),
    compiler_params=pltpu.CompilerParams(use_tc_tiling_on_sc=True,
                                         needs_layout_passes=False),
    out_type=jax.ShapeDtypeStruct((_NW * _LANES,), jnp.float32),
    scratch_types=[
        pltpu.VMEM((_ROWS_W,), jnp.int32),
        pltpu.VMEM((_N_CHUNKS, _IDX_CHUNK), jnp.int32),
        pltpu.VMEM((_ROWS_W // _LANES, _LANES), jnp.int32),
        pltpu.VMEM((2, _IDX_CHUNK, 128), jnp.float32),
        pltpu.VMEM((2, _IDX_CHUNK, 128), jnp.float32),
        pltpu.VMEM((_LANES,), jnp.float32),
        pltpu.SemaphoreType.DMA,
        pltpu.SemaphoreType.DMA,
    ],
)
def _center_loss_partials(x_hbm, labels_hbm, centers_hbm, out_hbm,
                          idx_v, row_v, off_v, rows_v, x_v, acc_v,
                          sem_x, sem_g):
    wid = lax.axis_index("s") * _NC + lax.axis_index("c")

    pltpu.sync_copy(
        labels_hbm.at[pl.ds(pl.multiple_of(wid * _ROWS_W, 8), _ROWS_W)],
        idx_v)
    for j in range(_N_CHUNKS):
        for k in range(_IDX_CHUNK // _LANES):
            v = idx_v[pl.ds(j * _IDX_CHUNK + k * _LANES, _LANES)]
            blk = lax.shift_right_logical(v, jnp.int32(_BN_LOG))
            row_v[j, pl.ds(k * _LANES, _LANES)] = (
                lax.shift_left(blk, jnp.int32(_QTR_LOG))
                + jnp.bitwise_and(v, jnp.int32(_QTR - 1)))
            off_v[j * (_IDX_CHUNK // _LANES) + k, pl.ds(0, _LANES)] = (
                lax.shift_left(
                    jnp.bitwise_and(
                        lax.shift_right_logical(v, jnp.int32(_QTR_LOG)),
                        jnp.int32(3)),
                    jnp.int32(5)))

    w15 = jnp.bitwise_and(wid, jnp.int32(15))
    xbase = (lax.shift_left(lax.shift_right_logical(wid, jnp.int32(4)),
                            jnp.int32(_QTR_LOG))
             + jnp.bitwise_and(w15, jnp.int32(3)) * jnp.int32(_ROWS_W))
    xoff = pl.multiple_of(
        lax.shift_left(lax.shift_right_logical(w15, jnp.int32(2)),
                       jnp.int32(5)), 32)

    def start_quarter(q):
        slot = q % 2
        xc = pltpu.async_copy(
            x_hbm.at[pl.ds(pl.multiple_of(xbase + q * _IDX_CHUNK, 8),
                           _IDX_CHUNK)],
            x_v.at[slot], sem_x)
        gc = pltpu.async_copy(centers_hbm.at[row_v.at[q]], rows_v.at[slot],
                              sem_g)
        return (xc, gc)

    def compute_quarter(q, acc):
        slot = q % 2

        def group_body(g, acc):
            off16 = off_v[q * _GROUPS_Q + g, pl.ds(0, _LANES)]
            for i in range(_LANES):
                r = g * _LANES + i
                off = pl.multiple_of(off16[i], 32)
                for half in range(2):
                    xv = x_v[slot, r, pl.ds(xoff + half * _LANES, _LANES)]
                    cv = rows_v[slot, r, pl.ds(off + half * _LANES, _LANES)]
                    xb = plsc.bitcast(xv, jnp.bfloat16)
                    cb = plsc.bitcast(cv, jnp.bfloat16)
                    d = xb - cb
                    da, db = plsc.unpack(
                        d, format=plsc.PackFormat.INTERLEAVED)
                    acc = acc + da * da
                    acc = acc + db * db
            return acc
        return lax.fori_loop(0, _GROUPS_Q, group_body, acc)

    acc = jnp.zeros((_LANES,), jnp.float32)
    pend = start_quarter(0)
    for q in range(_N_CHUNKS):
        nxt = start_quarter(q + 1) if q + 1 < _N_CHUNKS else None
        for c in pend:
            c.wait()
        acc = compute_quarter(q, acc)
        pend = nxt

    acc_v[...] = acc
    pltpu.sync_copy(
        acc_v,
        out_hbm.at[pl.ds(pl.multiple_of(wid * _LANES, 8), _LANES)])


def kernel(x, labels, centers):
    xp = _pack(x.T)
    cp = _pack(centers.T)
    labels_r = labels.astype(jnp.int32)
    partials = _center_loss_partials(xp, labels_r, cp)
    return jnp.sum(partials) * (1.0 / _BATCH)
